# Initial kernel scaffold; baseline (speedup 1.0000x reference)
import functools

import jax
import jax.numpy as jnp
from jax import lax
from jax.experimental import pallas as pl
from jax.experimental.pallas import tpu as pltpu
from jax.experimental.pallas import tpu_sc as plsc

_INFO = plsc.get_sparse_core_info()
_NC = _INFO.num_cores
_NS = _INFO.num_subcores
_NW = _NC * _NS

_CHUNK = 128


@functools.lru_cache(maxsize=None)
def _make_gather(V, D, N):
    assert N % (_NW * _CHUNK) == 0
    per_w = N // _NW
    nchunk = per_w // _CHUNK

    mesh = plsc.VectorSubcoreMesh(core_axis_name="c", subcore_axis_name="s")

    @functools.partial(
        pl.kernel,
        mesh=mesh,
        out_type=jax.ShapeDtypeStruct((N, D), jnp.float32),
        scratch_types=[
            pltpu.VMEM((nchunk, _CHUNK), jnp.int32),
            pltpu.VMEM((_CHUNK, D), jnp.float32),
            pltpu.SemaphoreType.DMA,
        ],
    )
    def gather_kernel(idx_hbm, table_hbm, out_hbm, idx_v, rows_v, gsem):
        wid = lax.axis_index("s") * _NC + lax.axis_index("c")
        base = wid * per_w
        pltpu.sync_copy(idx_hbm.at[wid], idx_v)

        def body(j, carry):
            pltpu.async_copy(table_hbm.at[idx_v.at[j]], rows_v, gsem).wait()
            pltpu.sync_copy(rows_v, out_hbm.at[pl.ds(base + j * _CHUNK, _CHUNK)])
            return carry

        lax.fori_loop(0, nchunk, body, 0)

    return gather_kernel


def kernel(inputs, embedding_table):
    B, H = inputs.shape
    V, D = embedding_table.shape
    N = B * H
    idx = inputs.reshape(N).astype(jnp.int32).reshape(_NW, N // _NW // _CHUNK, _CHUNK)
    out = _make_gather(V, D, N)(idx, embedding_table)
    return out.reshape(B, H, D)


# speedup vs baseline: 4.0916x; 4.0916x over previous
import functools

import jax
import jax.numpy as jnp
from jax import lax
from jax.experimental import pallas as pl
from jax.experimental.pallas import tpu as pltpu
from jax.experimental.pallas import tpu_sc as plsc

_INFO = plsc.get_sparse_core_info()
_NC = _INFO.num_cores
_NS = _INFO.num_subcores
_NW = _NC * _NS

_CHUNK = 128


@functools.lru_cache(maxsize=None)
def _make_gather(V, D, N):
    assert N % (_NW * _CHUNK) == 0
    per_w = N // _NW
    nchunk = per_w // _CHUNK

    mesh = plsc.VectorSubcoreMesh(core_axis_name="c", subcore_axis_name="s")

    @functools.partial(
        pl.kernel,
        mesh=mesh,
        out_type=jax.ShapeDtypeStruct((N, D), jnp.float32),
        scratch_types=[
            pltpu.VMEM((nchunk, _CHUNK), jnp.int32),
            pltpu.VMEM((_CHUNK, D), jnp.float32),
            pltpu.SemaphoreType.DMA,
        ],
        compiler_params=pltpu.CompilerParams(use_tc_tiling_on_sc=False),
    )
    def gather_kernel(idx_hbm, table_hbm, out_hbm, idx_v, rows_v, gsem):
        wid = lax.axis_index("s") * _NC + lax.axis_index("c")
        base = wid * per_w
        pltpu.sync_copy(idx_hbm.at[wid], idx_v)

        def body(j, carry):
            pltpu.async_copy(table_hbm.at[idx_v.at[j]], rows_v, gsem).wait()
            pltpu.sync_copy(rows_v, out_hbm.at[pl.ds(base + j * _CHUNK, _CHUNK)])
            return carry

        lax.fori_loop(0, nchunk, body, 0)

    return gather_kernel


def kernel(inputs, embedding_table):
    B, H = inputs.shape
    V, D = embedding_table.shape
    N = B * H
    idx = inputs.reshape(N).astype(jnp.int32).reshape(_NW, N // _NW // _CHUNK, _CHUNK)
    out = _make_gather(V, D, N)(idx, embedding_table)
    return out.reshape(B, H, D)


# speedup vs baseline: 4.6052x; 1.1255x over previous
import functools

import jax
import jax.numpy as jnp
from jax import lax
from jax.experimental import pallas as pl
from jax.experimental.pallas import tpu as pltpu
from jax.experimental.pallas import tpu_sc as plsc

_INFO = plsc.get_sparse_core_info()
_NC = _INFO.num_cores
_NS = _INFO.num_subcores
_NW = _NC * _NS

_CHUNK = 128


@functools.lru_cache(maxsize=None)
def _make_gather(V, D, N):
    assert N % (_NW * _CHUNK) == 0
    per_w = N // _NW
    nchunk = per_w // _CHUNK

    mesh = plsc.VectorSubcoreMesh(core_axis_name="c", subcore_axis_name="s")

    K = 5
    ngroups = nchunk // K
    assert nchunk % K == 0 and ngroups % 2 == 0

    @functools.partial(
        pl.kernel,
        mesh=mesh,
        out_type=jax.ShapeDtypeStruct((N, D), jnp.float32),
        scratch_types=[
            pltpu.VMEM((nchunk, _CHUNK), jnp.int32),
            pltpu.VMEM((2 * K, _CHUNK, D), jnp.float32),
            pltpu.SemaphoreType.DMA,
            pltpu.SemaphoreType.DMA,
        ],
        compiler_params=pltpu.CompilerParams(use_tc_tiling_on_sc=False),
    )
    def gather_kernel(idx_hbm, table_hbm, out_hbm, idx_v, rows_v, gsem, osem):
        wid = lax.axis_index("s") * _NC + lax.axis_index("c")
        base = wid * per_w
        pltpu.sync_copy(idx_hbm.at[wid], idx_v)

        def fire_gathers(g, setoff):
            return [
                pltpu.async_copy(
                    table_hbm.at[idx_v.at[g * K + b]], rows_v.at[setoff + b], gsem
                )
                for b in range(K)
            ]

        def fire_writes(g, setoff):
            return [
                pltpu.async_copy(
                    rows_v.at[setoff + b],
                    out_hbm.at[pl.ds(base + (g * K + b) * _CHUNK, _CHUNK)],
                    osem,
                )
                for b in range(K)
            ]

        def drain(descs):
            for d in descs:
                d.wait()

        ga = fire_gathers(0, 0)
        drain(ga)
        gb = fire_gathers(1, K)
        wa = fire_writes(0, 0)

        def body(p, carry):
            g = 2 * p + 1
            drain(gb)
            drain(wa)
            fire_gathers(g + 1, 0)
            wb = fire_writes(g, K)
            drain(ga)
            drain(wb)
            fire_gathers(g + 2, K)
            fire_writes(g + 1, 0)
            return carry

        lax.fori_loop(0, (ngroups - 2) // 2, body, 0)

        drain(gb)
        drain(wa)
        wlast = fire_writes(ngroups - 1, K)
        drain(wlast)

    return gather_kernel


def kernel(inputs, embedding_table):
    B, H = inputs.shape
    V, D = embedding_table.shape
    N = B * H
    idx = inputs.reshape(N).astype(jnp.int32).reshape(_NW, N // _NW // _CHUNK, _CHUNK)
    out = _make_gather(V, D, N)(idx, embedding_table)
    return out.reshape(B, H, D)


# speedup vs baseline: 4.9639x; 1.0779x over previous
import functools

import jax
import jax.numpy as jnp
from jax import lax
from jax.experimental import pallas as pl
from jax.experimental.pallas import tpu as pltpu
from jax.experimental.pallas import tpu_sc as plsc

_INFO = plsc.get_sparse_core_info()
_NC = _INFO.num_cores
_NS = _INFO.num_subcores
_NW = _NC * _NS


@functools.lru_cache(maxsize=None)
def _make_gather(V, D, B, H):
    assert D % _NW == 0 or _NW % D == 0
    npass = D // _NW
    assert npass * _NW == D
    BC = B // 128
    L = 16

    mesh = plsc.VectorSubcoreMesh(core_axis_name="c", subcore_axis_name="s")

    @functools.partial(
        pl.kernel,
        mesh=mesh,
        out_type=jax.ShapeDtypeStruct((H, D // 8, BC, 8, 128), jnp.float32),
        scratch_types=[
            pltpu.VMEM((V,), jnp.float32),
            pltpu.VMEM((2, BC, 128), jnp.int32),
            pltpu.VMEM((2, BC, 128), jnp.float32),
            pltpu.VMEM_SHARED((H, BC, 128), jnp.int32),
            pltpu.SemaphoreType.DMA,
        ],
        compiler_params=pltpu.CompilerParams(
            use_tc_tiling_on_sc=False, needs_layout_passes=False
        ),
    )
    def gather_kernel(idx_hbm, table_hbm, out_hbm, tbl_v, idx_v, row_v,
                      idx_sh, osem):
        cid = lax.axis_index("c")
        sid = lax.axis_index("s")
        worker = sid * _NC + cid

        @pl.when(sid == 0)
        def _():
            pltpu.sync_copy(idx_hbm, idx_sh)

        plsc.subcore_barrier()

        def compute_row(slot):
            for r in range(BC):
                for q in range(8):
                    iv = idx_v[slot, r, pl.ds(q * L, L)]
                    vals = plsc.load_gather(tbl_v, [iv])
                    row_v[slot, r, pl.ds(q * L, L)] = vals

        for p in range(npass):
            e = p * _NW + worker
            er = e // 8
            es = lax.rem(e, 8)
            pltpu.sync_copy(table_hbm.at[e], tbl_v)

            def do_row(h, slot):
                pltpu.sync_copy(idx_sh.at[h], idx_v.at[slot])
                compute_row(slot)
                return pltpu.async_copy(
                    row_v.at[slot], out_hbm.at[h, er, slice(None), es], osem
                )

            w0 = do_row(0, 0)
            w1 = do_row(1, 1)

            def body(t, carry):
                h = 2 * t + 2
                w0.wait()
                do_row(h, 0)
                w1.wait()
                do_row(h + 1, 1)
                return carry

            lax.fori_loop(0, (H - 2) // 2, body, 0)
            w0.wait()
            w1.wait()

    return gather_kernel


def kernel(inputs, embedding_table):
    B, H = inputs.shape
    V, D = embedding_table.shape
    idx_t = jnp.transpose(inputs).astype(jnp.int32).reshape(H, B // 128, 128)
    tab_t = jnp.transpose(embedding_table)
    out5 = _make_gather(V, D, B, H)(idx_t, tab_t)
    return jnp.transpose(out5, (2, 4, 0, 1, 3)).reshape(B, H, D)


# speedup vs baseline: 8.7333x; 1.7594x over previous
import functools

import jax
import jax.numpy as jnp
from jax import lax
from jax.experimental import pallas as pl
from jax.experimental.pallas import tpu as pltpu
from jax.experimental.pallas import tpu_sc as plsc

_INFO = plsc.get_sparse_core_info()
_NC = _INFO.num_cores
_NS = _INFO.num_subcores
_NW = _NC * _NS


@functools.lru_cache(maxsize=None)
def _make_gather(V, D, B, H):
    assert D % _NW == 0 or _NW % D == 0
    npass = D // _NW
    assert npass * _NW == D
    BC = B // 128
    L = 16

    mesh = plsc.VectorSubcoreMesh(core_axis_name="c", subcore_axis_name="s")

    @functools.partial(
        pl.kernel,
        mesh=mesh,
        out_type=jax.ShapeDtypeStruct((H, D // 8, BC, 8, 128), jnp.float32),
        scratch_types=[
            pltpu.VMEM((V,), jnp.float32),
            pltpu.VMEM((2, BC, 128), jnp.int32),
            pltpu.VMEM((2, BC, 128), jnp.float32),
            pltpu.VMEM_SHARED((H, BC, 128), jnp.int32),
            pltpu.SemaphoreType.DMA,
        ],
        compiler_params=pltpu.CompilerParams(
            use_tc_tiling_on_sc=False, needs_layout_passes=False
        ),
    )
    def gather_kernel(idx_hbm, table_hbm, out_hbm, tbl_v, idx_v, row_v,
                      idx_sh, osem):
        cid = lax.axis_index("c")
        sid = lax.axis_index("s")
        worker = sid * _NC + cid

        @pl.when(sid == 0)
        def _():
            pltpu.sync_copy(idx_hbm, idx_sh)

        plsc.subcore_barrier()

        def compute_row(slot):
            for r in range(0, BC, 2):
                ivs = [
                    idx_v[slot, r + q // 8, pl.ds((q % 8) * L, L)]
                    for q in range(16)
                ]
                vals = [plsc.load_gather(tbl_v, [iv]) for iv in ivs]
                for q in range(16):
                    row_v[slot, r + q // 8, pl.ds((q % 8) * L, L)] = vals[q]

        for p in range(npass):
            e = p * _NW + worker
            er = e // 8
            es = lax.rem(e, 8)
            pltpu.sync_copy(table_hbm.at[e], tbl_v)

            def do_row(h, slot):
                pltpu.sync_copy(idx_sh.at[h], idx_v.at[slot])
                compute_row(slot)
                return pltpu.async_copy(
                    row_v.at[slot], out_hbm.at[h, er, slice(None), es], osem
                )

            w0 = do_row(0, 0)
            w1 = do_row(1, 1)

            def body(t, carry):
                h = 2 * t + 2
                w0.wait()
                do_row(h, 0)
                w1.wait()
                do_row(h + 1, 1)
                return carry

            lax.fori_loop(0, (H - 2) // 2, body, 0)
            w0.wait()
            w1.wait()

    return gather_kernel


def kernel(inputs, embedding_table):
    B, H = inputs.shape
    V, D = embedding_table.shape
    idx_t = jnp.transpose(inputs).astype(jnp.int32).reshape(H, B // 128, 128)
    tab_t = jnp.transpose(embedding_table)
    out5 = _make_gather(V, D, B, H)(idx_t, tab_t)
    return jnp.transpose(out5, (2, 4, 0, 1, 3)).reshape(B, H, D)


# speedup vs baseline: 11.4295x; 1.3087x over previous
import functools

import jax
import jax.numpy as jnp
from jax import lax
from jax.experimental import pallas as pl
from jax.experimental.pallas import tpu as pltpu
from jax.experimental.pallas import tpu_sc as plsc

_INFO = plsc.get_sparse_core_info()
_NC = _INFO.num_cores
_NS = _INFO.num_subcores
_NW = _NC * _NS


@functools.lru_cache(maxsize=None)
def _make_gather(V, D, B, H):
    assert D % _NW == 0 or _NW % D == 0
    npass = D // _NW
    assert npass * _NW == D
    BC = B // 128
    L = 16

    mesh = plsc.VectorSubcoreMesh(core_axis_name="c", subcore_axis_name="s")

    @functools.partial(
        pl.kernel,
        mesh=mesh,
        out_type=jax.ShapeDtypeStruct((H, D // 8, BC, 8, 128), jnp.float32),
        scratch_types=[
            pltpu.VMEM((V,), jnp.float32),
            pltpu.VMEM((2, BC, 128), jnp.int32),
            pltpu.VMEM((2, BC, 128), jnp.float32),
            pltpu.VMEM_SHARED((H, BC, 128), jnp.int32),
            pltpu.SemaphoreType.DMA,
        ],
        compiler_params=pltpu.CompilerParams(
            use_tc_tiling_on_sc=True, needs_layout_passes=False
        ),
    )
    def gather_kernel(idx_hbm, table_hbm, out_hbm, tbl_v, idx_v, row_v,
                      idx_sh, osem):
        cid = lax.axis_index("c")
        sid = lax.axis_index("s")
        worker = sid * _NC + cid

        @pl.when(sid == 0)
        def _():
            pltpu.sync_copy(idx_hbm, idx_sh)

        plsc.subcore_barrier()

        def compute_row(slot):
            for r in range(0, BC, 2):
                ivs = [
                    idx_v[slot, r + q // 8, pl.ds((q % 8) * L, L)]
                    for q in range(16)
                ]
                vals = [plsc.load_gather(tbl_v, [iv]) for iv in ivs]
                for q in range(16):
                    row_v[slot, r + q // 8, pl.ds((q % 8) * L, L)] = vals[q]

        for p in range(npass):
            e = p * _NW + worker
            er = e // 8
            es = lax.rem(e, 8)
            pltpu.sync_copy(table_hbm.at[e], tbl_v)

            def do_row(h, slot):
                pltpu.sync_copy(idx_sh.at[h], idx_v.at[slot])
                compute_row(slot)
                return pltpu.async_copy(
                    row_v.at[slot], out_hbm.at[h, er, slice(None), es], osem
                )

            w0 = do_row(0, 0)
            w1 = do_row(1, 1)

            def body(t, carry):
                h = 2 * t + 2
                w0.wait()
                do_row(h, 0)
                w1.wait()
                do_row(h + 1, 1)
                return carry

            lax.fori_loop(0, (H - 2) // 2, body, 0)
            w0.wait()
            w1.wait()

    return gather_kernel


def kernel(inputs, embedding_table):
    B, H = inputs.shape
    V, D = embedding_table.shape
    idx_t = jnp.transpose(inputs).astype(jnp.int32).reshape(H, B // 128, 128)
    tab_t = jnp.transpose(embedding_table)
    out5 = _make_gather(V, D, B, H)(idx_t, tab_t)
    return jnp.transpose(out5, (2, 4, 0, 1, 3)).reshape(B, H, D)
